# Initial kernel scaffold; baseline (speedup 1.0000x reference)
import jax
import jax.numpy as jnp
from jax.experimental import pallas as pl


def kernel(x, edge_index, W1, b1, W2, b2, Wlin, blin):
    raise NotImplementedError("write your pallas kernel here")


# speedup vs baseline: 35.3936x; 35.3936x over previous
import jax
import jax.numpy as jnp
from jax import lax
from jax.experimental import pallas as pl
from jax.experimental.pallas import tpu as pltpu
from jax.experimental.pallas import tpu_sc as plsc

N = 10000
E = 320000
DIN = 128
D = 16
NC = 2
NS = 16
NW = NC * NS
EPT = E // NW
CH = 125
NCH = EPT // CH
NP = 10240
NPS = NP // NS
RB = 2000

_mesh = plsc.VectorSubcoreMesh(core_axis_name="c", subcore_axis_name="s")


def _deg_body(dst_hbm, out_hbm, dst_v, ones_v, acc):
    c = lax.axis_index("c")
    s = lax.axis_index("s")
    wid = c * NS + s
    pltpu.sync_copy(dst_hbm.at[wid], dst_v)
    one_row = jnp.full((D,), 1.0, jnp.float32)
    for i in range(128):
        ones_v[i, :] = one_row
    r0 = s * NPS
    for k in range(NPS // 128):
        pltpu.sync_copy(ones_v, acc.at[pl.ds(r0 + k * 128, 128)])
    plsc.subcore_barrier()

    def chunk(j, carry):
        pltpu.sync_copy(ones_v.at[pl.ds(0, CH)], acc.at[dst_v.at[j]], add=True)
        return carry

    lax.fori_loop(0, NCH, chunk, 0)
    plsc.subcore_barrier()
    pltpu.sync_copy(acc.at[pl.ds(r0, NPS)], out_hbm.at[c, pl.ds(r0, NPS)])


_deg_kernel = pl.kernel(
    _deg_body,
    out_type=jax.ShapeDtypeStruct((NC, NP, D), jnp.float32),
    mesh=_mesh,
    compiler_params=pltpu.CompilerParams(use_tc_tiling_on_sc=False),
    scratch_types=[
        pltpu.VMEM((NCH, CH), jnp.int32),
        pltpu.VMEM((128, D), jnp.float32),
        pltpu.VMEM_SHARED((NP, D), jnp.float32),
    ],
)


def _agg_body(tbl_hbm, src_hbm, dst_hbm, out_hbm, src_v, dst_v, rows_v, acc):
    c = lax.axis_index("c")
    s = lax.axis_index("s")
    wid = c * NS + s
    pltpu.sync_copy(src_hbm.at[wid], src_v)
    pltpu.sync_copy(dst_hbm.at[wid], dst_v)
    r0 = s * NPS
    pltpu.sync_copy(tbl_hbm.at[pl.ds(r0, NPS)], acc.at[pl.ds(r0, NPS)])
    plsc.subcore_barrier()

    def chunk(j, carry):
        pltpu.sync_copy(tbl_hbm.at[src_v.at[j]], rows_v)
        pltpu.sync_copy(rows_v, acc.at[dst_v.at[j]], add=True)
        return carry

    lax.fori_loop(0, NCH, chunk, 0)
    plsc.subcore_barrier()
    pltpu.sync_copy(acc.at[pl.ds(r0, NPS)], out_hbm.at[c, pl.ds(r0, NPS)])


_agg_kernel = pl.kernel(
    _agg_body,
    out_type=jax.ShapeDtypeStruct((NC, NP, D), jnp.float32),
    mesh=_mesh,
    compiler_params=pltpu.CompilerParams(use_tc_tiling_on_sc=False),
    scratch_types=[
        pltpu.VMEM((NCH, CH), jnp.int32),
        pltpu.VMEM((NCH, CH), jnp.int32),
        pltpu.VMEM((CH, D), jnp.float32),
        pltpu.VMEM_SHARED((NP, D), jnp.float32),
    ],
)


def _tc1_body(x_ref, w_ref, dacc_ref, g_ref, dis_ref):
    h = jnp.dot(
        x_ref[...].astype(jnp.bfloat16),
        w_ref[...].astype(jnp.bfloat16),
        preferred_element_type=jnp.float32,
    )
    deg = dacc_ref[0] + dacc_ref[1] - 1.0
    dis = 1.0 / jnp.sqrt(deg)
    dis_ref[...] = dis
    g_ref[...] = h * dis


def _tc1(x, W1, dacc):
    return pl.pallas_call(
        _tc1_body,
        grid=(N // RB,),
        in_specs=[
            pl.BlockSpec((RB, DIN), lambda i: (i, 0)),
            pl.BlockSpec((DIN, D), lambda i: (0, 0)),
            pl.BlockSpec((NC, RB, D), lambda i: (0, i, 0)),
        ],
        out_specs=(
            pl.BlockSpec((RB, D), lambda i: (i, 0)),
            pl.BlockSpec((RB, D), lambda i: (i, 0)),
        ),
        out_shape=(
            jax.ShapeDtypeStruct((NP, D), jnp.float32),
            jax.ShapeDtypeStruct((N, D), jnp.float32),
        ),
    )(x, W1, dacc)


def _tc2_body(acc_ref, g_ref, dis_ref, w_ref, b_ref, out_ref):
    z = dis_ref[...] * (acc_ref[0] + acc_ref[1] - g_ref[...]) + b_ref[...]
    z = jnp.maximum(z, 0.0)
    out_ref[...] = (
        jnp.dot(
            z.astype(jnp.bfloat16),
            w_ref[...].astype(jnp.bfloat16),
            preferred_element_type=jnp.float32,
        )
        * dis_ref[...]
    )


def _tc2(acc, g, dis, W2, b1):
    return pl.pallas_call(
        _tc2_body,
        grid=(N // RB,),
        in_specs=[
            pl.BlockSpec((NC, RB, D), lambda i: (0, i, 0)),
            pl.BlockSpec((RB, D), lambda i: (i, 0)),
            pl.BlockSpec((RB, D), lambda i: (i, 0)),
            pl.BlockSpec((D, D), lambda i: (0, 0)),
            pl.BlockSpec((1, D), lambda i: (0, 0)),
        ],
        out_specs=pl.BlockSpec((RB, D), lambda i: (i, 0)),
        out_shape=jax.ShapeDtypeStruct((NP, D), jnp.float32),
    )(acc, g, dis, W2, b1)


def _tc3_body(acc_ref, g_ref, dis_ref, b_ref, wl_ref, bl_ref, out_ref):
    z = dis_ref[...] * (acc_ref[0] + acc_ref[1] - g_ref[...]) + b_ref[...]
    z = jnp.maximum(z, 0.0)
    zb = z.astype(jnp.bfloat16).astype(jnp.float32)
    wlb = wl_ref[...].astype(jnp.bfloat16).astype(jnp.float32)
    out_ref[...] = jnp.sum(zb * wlb, axis=1, keepdims=True) + bl_ref[...]


def _tc3(acc, g, dis, b2, wl, bl):
    return pl.pallas_call(
        _tc3_body,
        grid=(N // RB,),
        in_specs=[
            pl.BlockSpec((NC, RB, D), lambda i: (0, i, 0)),
            pl.BlockSpec((RB, D), lambda i: (i, 0)),
            pl.BlockSpec((RB, D), lambda i: (i, 0)),
            pl.BlockSpec((1, D), lambda i: (0, 0)),
            pl.BlockSpec((1, D), lambda i: (0, 0)),
            pl.BlockSpec((1, 1), lambda i: (0, 0)),
        ],
        out_specs=pl.BlockSpec((RB, 1), lambda i: (i, 0)),
        out_shape=jax.ShapeDtypeStruct((N, 1), jnp.float32),
    )(acc, g, dis, b2, wl, bl)


def kernel(x, edge_index, W1, b1, W2, b2, Wlin, blin):
    src = edge_index[0].reshape(NW, NCH, CH)
    dst = edge_index[1].reshape(NW, NCH, CH)
    dacc = _deg_kernel(dst)
    g1, dis = _tc1(x, W1, dacc)
    acc1 = _agg_kernel(g1, src, dst)
    g2 = _tc2(acc1, g1, dis, W2, b1.reshape(1, D))
    acc2 = _agg_kernel(g2, src, dst)
    out = _tc3(acc2, g2, dis, b2.reshape(1, D), Wlin.reshape(1, D), blin.reshape(1, 1))
    return jnp.squeeze(out)


# speedup vs baseline: 63.1477x; 1.7842x over previous
import jax
import jax.numpy as jnp
from jax import lax
from jax.experimental import pallas as pl
from jax.experimental.pallas import tpu as pltpu
from jax.experimental.pallas import tpu_sc as plsc

N = 10000
E = 320000
DIN = 128
D = 16
NC = 2
NS = 16
NW = NC * NS
EPT = E // NW
CH = 1000
NCH = EPT // CH
NP = 10240
NPS = NP // NS
NPQ = NP // D
RB = 2000
RB2 = 2048

_mesh = plsc.VectorSubcoreMesh(core_axis_name="c", subcore_axis_name="s")


def _deg_body(dst_hbm, out_hbm, dst_v, hist, sem0):
    c = lax.axis_index("c")
    s = lax.axis_index("s")
    wid = c * NS + s
    stage = pltpu.make_async_copy(dst_hbm.at[wid], dst_v, sem0)
    stage.start()
    zero_row = jnp.zeros((D,), jnp.float32)

    def zbody(i, carry):
        for k in range(5):
            hist[i * 5 + k, :] = zero_row
        return carry

    lax.fori_loop(0, NPQ // 5, zbody, 0)
    stage.wait()
    ones = jnp.full((D,), 1.0, jnp.float32)

    def chunk(i, carry):
        for k in range(5):
            idx = dst_v[pl.ds((i * 5 + k) * D, D)]
            plsc.addupdate_scatter(
                hist, [idx >> 4, idx & 15], ones
            )
        return carry

    lax.fori_loop(0, EPT // D // 5, chunk, 0)
    pltpu.sync_copy(hist, out_hbm.at[wid])


_deg_kernel = pl.kernel(
    _deg_body,
    out_type=jax.ShapeDtypeStruct((NW, NPQ, D), jnp.float32),
    mesh=_mesh,
    compiler_params=pltpu.CompilerParams(
        use_tc_tiling_on_sc=False, needs_layout_passes=False
    ),
    scratch_types=[
        pltpu.VMEM((EPT,), jnp.int32),
        pltpu.VMEM((NPQ, D), jnp.float32),
        pltpu.SemaphoreType.DMA,
    ],
)


def _agg_body(tbl_hbm, src_hbm, dst_hbm, out_hbm, src_v, dst_v, rows0, rows1, acc,
              sem0, sem1, sem2, sem3):
    c = lax.axis_index("c")
    s = lax.axis_index("s")
    wid = c * NS + s
    stage_s = pltpu.make_async_copy(src_hbm.at[wid], src_v, sem0)
    stage_s.start()
    stage_d = pltpu.make_async_copy(dst_hbm.at[wid], dst_v, sem1)
    stage_d.start()
    r0 = s * NPS
    pltpu.sync_copy(tbl_hbm.at[pl.ds(r0, NPS)], acc.at[pl.ds(r0, NPS)])
    stage_s.wait()
    stage_d.wait()
    plsc.subcore_barrier()

    rows = (rows0, rows1)
    gsem = (sem0, sem1)
    ssem = (sem2, sem3)
    gd = [None] * NCH
    sd = [None] * NCH
    gd[0] = pltpu.async_copy(tbl_hbm.at[src_v.at[0]], rows[0], gsem[0])
    for j in range(NCH):
        if j + 1 < NCH:
            if j >= 1:
                sd[j - 1].wait()
            gd[j + 1] = pltpu.async_copy(
                tbl_hbm.at[src_v.at[j + 1]], rows[(j + 1) % 2], gsem[(j + 1) % 2]
            )
        gd[j].wait()
        sd[j] = pltpu.async_copy(
            rows[j % 2], acc.at[dst_v.at[j]], ssem[j % 2], add=True
        )
    sd[NCH - 2].wait()
    sd[NCH - 1].wait()
    plsc.subcore_barrier()
    pltpu.sync_copy(acc.at[pl.ds(r0, NPS)], out_hbm.at[c, pl.ds(r0, NPS)])


_agg_kernel = pl.kernel(
    _agg_body,
    out_type=jax.ShapeDtypeStruct((NC, NP, D), jnp.float32),
    mesh=_mesh,
    compiler_params=pltpu.CompilerParams(use_tc_tiling_on_sc=False),
    scratch_types=[
        pltpu.VMEM((NCH, CH), jnp.int32),
        pltpu.VMEM((NCH, CH), jnp.int32),
        pltpu.VMEM((CH, D), jnp.float32),
        pltpu.VMEM((CH, D), jnp.float32),
        pltpu.VMEM_SHARED((NP, D), jnp.float32),
        pltpu.SemaphoreType.DMA,
        pltpu.SemaphoreType.DMA,
        pltpu.SemaphoreType.DMA,
        pltpu.SemaphoreType.DMA,
    ],
)


def _tc1_body(x_ref, w_ref, dacc_ref, g_ref, dis_ref):
    h = jnp.dot(
        x_ref[...].astype(jnp.bfloat16),
        w_ref[...].astype(jnp.bfloat16),
        preferred_element_type=jnp.float32,
    )
    deg = jnp.sum(dacc_ref[...], axis=0) + 1.0
    disq = 1.0 / jnp.sqrt(deg)
    rowid = lax.broadcasted_iota(jnp.int32, (RB2, RB2 // D), 0)
    colid = lax.broadcasted_iota(jnp.int32, (RB2, RB2 // D), 1)
    sel = ((rowid >> 4) == colid).astype(jnp.float32)
    pick = jnp.dot(sel, disq, preferred_element_type=jnp.float32,
                   precision=lax.Precision.HIGHEST)
    lane = lax.broadcasted_iota(jnp.int32, (RB2, D), 1)
    rmod = lax.broadcasted_iota(jnp.int32, (RB2, D), 0) & 15
    dis_col = jnp.sum(jnp.where(lane == rmod, pick, 0.0), axis=1, keepdims=True)
    dis_ref[...] = jnp.broadcast_to(dis_col, (RB2, D))
    g_ref[...] = h * dis_col


def _tc1(x, W1, dacc):
    return pl.pallas_call(
        _tc1_body,
        grid=(NP // RB2,),
        in_specs=[
            pl.BlockSpec((RB2, DIN), lambda i: (i, 0)),
            pl.BlockSpec((DIN, D), lambda i: (0, 0)),
            pl.BlockSpec((NW, RB2 // D, D), lambda i: (0, i, 0)),
        ],
        out_specs=(
            pl.BlockSpec((RB2, D), lambda i: (i, 0)),
            pl.BlockSpec((RB2, D), lambda i: (i, 0)),
        ),
        out_shape=(
            jax.ShapeDtypeStruct((NP, D), jnp.float32),
            jax.ShapeDtypeStruct((NP, D), jnp.float32),
        ),
    )(x, W1, dacc)


def _tc2_body(acc_ref, g_ref, dis_ref, w_ref, b_ref, out_ref):
    z = dis_ref[...] * (acc_ref[0] + acc_ref[1] - g_ref[...]) + b_ref[...]
    z = jnp.maximum(z, 0.0)
    out_ref[...] = (
        jnp.dot(
            z.astype(jnp.bfloat16),
            w_ref[...].astype(jnp.bfloat16),
            preferred_element_type=jnp.float32,
        )
        * dis_ref[...]
    )


def _tc2(acc, g, dis, W2, b1):
    return pl.pallas_call(
        _tc2_body,
        grid=(N // RB,),
        in_specs=[
            pl.BlockSpec((NC, RB, D), lambda i: (0, i, 0)),
            pl.BlockSpec((RB, D), lambda i: (i, 0)),
            pl.BlockSpec((RB, D), lambda i: (i, 0)),
            pl.BlockSpec((D, D), lambda i: (0, 0)),
            pl.BlockSpec((1, D), lambda i: (0, 0)),
        ],
        out_specs=pl.BlockSpec((RB, D), lambda i: (i, 0)),
        out_shape=jax.ShapeDtypeStruct((NP, D), jnp.float32),
    )(acc, g, dis, W2, b1)


def _tc3_body(acc_ref, g_ref, dis_ref, b_ref, wl_ref, bl_ref, out_ref):
    z = dis_ref[...] * (acc_ref[0] + acc_ref[1] - g_ref[...]) + b_ref[...]
    z = jnp.maximum(z, 0.0)
    zb = z.astype(jnp.bfloat16).astype(jnp.float32)
    wlb = wl_ref[...].astype(jnp.bfloat16).astype(jnp.float32)
    out_ref[...] = jnp.sum(zb * wlb, axis=1, keepdims=True) + bl_ref[...]


def _tc3(acc, g, dis, b2, wl, bl):
    return pl.pallas_call(
        _tc3_body,
        grid=(N // RB,),
        in_specs=[
            pl.BlockSpec((NC, RB, D), lambda i: (0, i, 0)),
            pl.BlockSpec((RB, D), lambda i: (i, 0)),
            pl.BlockSpec((RB, D), lambda i: (i, 0)),
            pl.BlockSpec((1, D), lambda i: (0, 0)),
            pl.BlockSpec((1, D), lambda i: (0, 0)),
            pl.BlockSpec((1, 1), lambda i: (0, 0)),
        ],
        out_specs=pl.BlockSpec((RB, 1), lambda i: (i, 0)),
        out_shape=jax.ShapeDtypeStruct((N, 1), jnp.float32),
    )(acc, g, dis, b2, wl, bl)


def kernel(x, edge_index, W1, b1, W2, b2, Wlin, blin):
    src = edge_index[0].reshape(NW, NCH, CH)
    dst = edge_index[1].reshape(NW, NCH, CH)
    dst_d = edge_index[1].reshape(NW, EPT)
    dacc = _deg_kernel(dst_d)
    g1, dis = _tc1(x, W1, dacc)
    acc1 = _agg_kernel(g1, src, dst)
    g2 = _tc2(acc1, g1, dis, W2, b1.reshape(1, D))
    acc2 = _agg_kernel(g2, src, dst)
    out = _tc3(acc2, g2, dis, b2.reshape(1, D), Wlin.reshape(1, D), blin.reshape(1, 1))
    return jnp.squeeze(out)
